# Initial kernel scaffold; baseline (speedup 1.0000x reference)
import jax
import jax.numpy as jnp
from jax.experimental import pallas as pl


def kernel(x, Wg1, W1, b1, Wg2, W2, b2):
    raise NotImplementedError("write your pallas kernel here")


# speedup vs baseline: 1.0460x; 1.0460x over previous
import functools

import jax
import jax.numpy as jnp
from jax import lax
from jax.experimental import pallas as pl
from jax.experimental.pallas import tpu as pltpu
from jax.experimental.pallas import tpu_sc as plsc

BLK = 128


def _router_body(cap, nexp, p_ref, gate_ref, gatem_ref, dsts_ref, dstg_ref,
                 cnt_ref):
    @pl.when(pl.program_id(0) == 0)
    def _init():
        cnt_ref[...] = jnp.zeros_like(cnt_ref)

    p = p_ref[...]
    gate = gate_ref[0]
    col = lax.broadcasted_iota(jnp.int32, p.shape, 1)
    idx = jnp.min(jnp.where(p == gate, col, nexp), axis=-1,
                  keepdims=True)
    me = (col == idx).astype(jnp.float32)
    blk = p.shape[0]
    r_i = lax.broadcasted_iota(jnp.int32, (blk, blk), 0)
    c_i = lax.broadcasted_iota(jnp.int32, (blk, blk), 1)
    tril = (c_i <= r_i).astype(jnp.float32)
    incl = jnp.dot(tril, me, preferred_element_type=jnp.float32)
    pos = (incl + cnt_ref[...]) * me
    cnt_ref[...] = cnt_ref[...] + jnp.sum(me, axis=0, keepdims=True)
    pos_t = jnp.sum(pos, axis=-1, keepdims=True)
    keep = pos_t <= cap
    slot = pos_t.astype(jnp.int32) - 1
    dst = idx * cap + slot
    gatem_ref[0] = jnp.where(keep, gate, 0.0)
    dsts_ref[0] = jnp.where(keep, dst, nexp * cap)
    dstg_ref[0] = jnp.where(keep, dst, idx * cap)


def _mm_body(x_ref, w_ref, b_ref, o_ref):
    o_ref[...] = (jnp.dot(x_ref[...], w_ref[0],
                          preferred_element_type=jnp.float32) + b_ref[0])


def _scale_body(g_ref, gate_ref, o_ref):
    gq = g_ref[...].astype(jnp.bfloat16).astype(jnp.float32)
    sq = gate_ref[0].astype(jnp.bfloat16).astype(jnp.float32)
    o_ref[...] = gq * sq


def _make_router(T, E, cap):
    nb = T // BLK
    return pl.pallas_call(
        functools.partial(_router_body, cap, E),
        grid=(nb,),
        in_specs=[pl.BlockSpec((BLK, E), lambda i: (i, 0)),
                  pl.BlockSpec((1, BLK, 1), lambda i: (i, 0, 0))],
        out_specs=[pl.BlockSpec((1, BLK, 1), lambda i: (i, 0, 0))] * 3,
        out_shape=[jax.ShapeDtypeStruct((nb, BLK, 1), jnp.float32),
                   jax.ShapeDtypeStruct((nb, BLK, 1), jnp.int32),
                   jax.ShapeDtypeStruct((nb, BLK, 1), jnp.int32)],
        scratch_shapes=[pltpu.VMEM((1, E), jnp.float32)],
    )


def _make_mm(E, cap, din, dout):
    return pl.pallas_call(
        _mm_body,
        grid=(E,),
        in_specs=[pl.BlockSpec((cap, din), lambda e: (e, 0)),
                  pl.BlockSpec((1, din, dout), lambda e: (e, 0, 0)),
                  pl.BlockSpec((1, 1, dout), lambda e: (e, 0, 0))],
        out_specs=pl.BlockSpec((cap, dout), lambda e: (e, 0)),
        out_shape=jax.ShapeDtypeStruct((E * cap, dout), jnp.float32),
        compiler_params=pltpu.CompilerParams(
            dimension_semantics=("arbitrary",)),
    )


def _make_scale(T, D):
    nb = T // BLK
    return pl.pallas_call(
        _scale_body,
        grid=(nb,),
        in_specs=[pl.BlockSpec((BLK, D), lambda i: (i, 0)),
                  pl.BlockSpec((1, BLK, 1), lambda i: (i, 0, 0))],
        out_specs=pl.BlockSpec((BLK, D), lambda i: (i, 0)),
        out_shape=jax.ShapeDtypeStruct((T, D), jnp.float32),
    )


def _make_sc_scatter(T, D, nrows, tpw):
    mesh = plsc.VectorSubcoreMesh(core_axis_name="c", subcore_axis_name="s")
    nc = mesh.num_cores

    @functools.partial(
        pl.kernel, mesh=mesh,
        out_type=jax.ShapeDtypeStruct((nrows, D), jnp.float32),
        scratch_types=[pltpu.VMEM((tpw,), jnp.int32),
                       pltpu.VMEM((tpw, D), jnp.float32),
                       pltpu.SemaphoreType.DMA],
    )
    def k(src_hbm, idx_hbm, out_hbm, idx_v, rows_v, sem):
        wid = lax.axis_index("s") * nc + lax.axis_index("c")
        base = wid * tpw
        pltpu.sync_copy(idx_hbm.at[pl.ds(base, tpw)], idx_v)
        pltpu.sync_copy(src_hbm.at[pl.ds(base, tpw)], rows_v)
        pltpu.async_copy(rows_v, out_hbm.at[idx_v], sem).wait()

    return k


def _make_sc_gather(T, D, tpw):
    mesh = plsc.VectorSubcoreMesh(core_axis_name="c", subcore_axis_name="s")
    nc = mesh.num_cores

    @functools.partial(
        pl.kernel, mesh=mesh,
        out_type=jax.ShapeDtypeStruct((T, D), jnp.float32),
        scratch_types=[pltpu.VMEM((tpw,), jnp.int32),
                       pltpu.VMEM((tpw, D), jnp.float32),
                       pltpu.SemaphoreType.DMA],
    )
    def k(table_hbm, idx_hbm, out_hbm, idx_v, rows_v, sem):
        wid = lax.axis_index("s") * nc + lax.axis_index("c")
        base = wid * tpw
        pltpu.sync_copy(idx_hbm.at[pl.ds(base, tpw)], idx_v)
        pltpu.async_copy(table_hbm.at[idx_v], rows_v, sem).wait()
        pltpu.sync_copy(rows_v, out_hbm.at[pl.ds(base, tpw)])

    return k


def _moe_layer(xt, router_x, Wg, W, b, cap, tpw):
    T, _ = xt.shape
    E, din, dout = W.shape
    nb = T // BLK
    nrows = (E + 1) * cap
    logits = jnp.dot(router_x, Wg, preferred_element_type=jnp.float32)
    probs = jax.nn.softmax(logits, axis=-1)
    gate = jnp.max(probs, axis=-1).reshape(nb, BLK, 1)
    gate_m, dsts, dstg = _make_router(T, E, cap)(probs, gate)
    xe = _make_sc_scatter(T, din, nrows, tpw)(xt, dsts.reshape(T))
    he = _make_mm(E, cap, din, dout)(xe, W, b.reshape(E, 1, dout))
    g = _make_sc_gather(T, dout, tpw)(he, dstg.reshape(T))
    return _make_scale(T, dout)(g, gate_m)


def kernel(x, Wg1, W1, b1, Wg2, W2, b2):
    B, S, D = x.shape
    E = W1.shape[0]
    T = B * S
    cap = 2 * T // E
    info = plsc.get_sparse_core_info()
    tpw = T // (info.num_cores * info.num_subcores)

    xt = x.reshape(T, D)
    y1 = _moe_layer(xt, xt, Wg1, W1, b1, cap, tpw)
    a = jax.nn.gelu(y1)
    y = _moe_layer(a, a.astype(jnp.bfloat16), Wg2, W2, b2, cap, tpw)
    return y.reshape(B, S, D)
